# baseline (speedup 1.0000x reference)
import functools
import jax
import jax.numpy as jnp
from jax import lax
from jax.experimental import pallas as pl
from jax.experimental.pallas import tpu as pltpu
from jax.experimental.pallas import tpu_sc as plsc

_VOCAB = 1000
_DIM = 64
_B = 4096
_X_MAX = 100.0
_ALPHA = 0.75

_NC = 2
_NS = 16
_NW = _NC * _NS
_CHUNK = _B // _NW
_HALF = _CHUNK // 2
_L = 16

_SC_PARAMS = pltpu.CompilerParams(
    use_tc_tiling_on_sc=False,
    needs_layout_passes=False,
)
_MESH = plsc.VectorSubcoreMesh(core_axis_name="c", subcore_axis_name="s")


def _worker_base():
    wid = lax.axis_index("s") * _NC + lax.axis_index("c")
    return wid * _CHUNK


def _sc_embed_fn():
    @functools.partial(
        pl.kernel,
        mesh=_MESH,
        out_type=(
            jax.ShapeDtypeStruct((_B,), jnp.float32),
            jax.ShapeDtypeStruct((_B,), jnp.float32),
        ),
        scratch_types=[
            pltpu.VMEM((_HALF,), jnp.int32),
            pltpu.VMEM((_HALF,), jnp.int32),
            pltpu.VMEM((_HALF,), jnp.int32),
            pltpu.VMEM((_HALF,), jnp.int32),
            pltpu.VMEM((_HALF, _DIM), jnp.float32),
            pltpu.VMEM((_HALF, _DIM), jnp.float32),
            pltpu.VMEM((_HALF, _DIM), jnp.float32),
            pltpu.VMEM((_HALF, _DIM), jnp.float32),
            pltpu.VMEM((_CHUNK,), jnp.float32),
            pltpu.VMEM((_CHUNK,), jnp.float32),
            pltpu.VMEM((_CHUNK * _L,), jnp.float32),
            pltpu.VMEM((_CHUNK,), jnp.float32),
            pltpu.VMEM((_CHUNK,), jnp.float32),
            pltpu.SemaphoreType.DMA,
            pltpu.SemaphoreType.DMA,
            pltpu.SemaphoreType.DMA,
            pltpu.SemaphoreType.DMA,
        ],
        compiler_params=_SC_PARAMS,
    )
    def sc_kernel(words_hbm, targets_hbm, wtw_hbm, btb_hbm,
                  dot_out, c_out,
                  wv1, wv2, tv1, tv2, ew1, ew2, etw1, etw2, bg, tbg,
                  qflat, dotv, cv, sem0, sem1, sem2, sem3):
        base = _worker_base()
        i1 = pltpu.async_copy(words_hbm.at[pl.ds(base, _HALF)], wv1, sem0)
        i2 = pltpu.async_copy(targets_hbm.at[pl.ds(base, _HALF)], tv1, sem0)
        i3 = pltpu.async_copy(
            words_hbm.at[pl.ds(base + _HALF, _HALF)], wv2, sem0)
        i4 = pltpu.async_copy(
            targets_hbm.at[pl.ds(base + _HALF, _HALF)], tv2, sem0)
        for cp in (i1, i2, i3, i4):
            cp.wait()
        for i in range(_HALF // _L):
            s = pl.ds(i * _L, _L)
            tv1[s] = tv1[s] + _VOCAB
            tv2[s] = tv2[s] + _VOCAB
        cp1a = pltpu.async_copy(wtw_hbm.at[wv1], ew1, sem1)
        cp1b = pltpu.async_copy(wtw_hbm.at[tv1], etw1, sem1)
        cp2a = pltpu.async_copy(wtw_hbm.at[wv2], ew2, sem2)
        cp2b = pltpu.async_copy(wtw_hbm.at[tv2], etw2, sem2)
        cp3a = pltpu.async_copy(btb_hbm.at[wv1], bg.at[pl.ds(0, _HALF)], sem3)
        cp3b = pltpu.async_copy(
            btb_hbm.at[wv2], bg.at[pl.ds(_HALF, _HALF)], sem3)
        cp3c = pltpu.async_copy(
            btb_hbm.at[tv1], tbg.at[pl.ds(0, _HALF)], sem3)
        cp3d = pltpu.async_copy(
            btb_hbm.at[tv2], tbg.at[pl.ds(_HALF, _HALF)], sem3)

        def fold_half(ew, etw, qbase):
            @plsc.parallel_loop(0, _HALF, unroll=4)
            def fold_body(k):
                s = pl.ds(0, _L)
                acc = ew[k, s] * etw[k, s]
                for j in range(1, _DIM // _L):
                    s = pl.ds(j * _L, _L)
                    acc = acc + ew[k, s] * etw[k, s]
                qflat[pl.ds(qbase + k * _L, _L)] = acc

        cp1a.wait()
        cp1b.wait()
        fold_half(ew1, etw1, 0)
        cp2a.wait()
        cp2b.wait()
        fold_half(ew2, etw2, _HALF * _L)

        lanebase = lax.iota(jnp.int32, _L) * _L

        @plsc.parallel_loop(0, _CHUNK // _L)
        def red_group(g):
            gb = g * (_L * _L)
            acc = plsc.load_gather(qflat, [lanebase + gb])
            for d in range(1, _L):
                acc = acc + plsc.load_gather(qflat, [lanebase + (gb + d)])
            dotv[pl.ds(g * _L, _L)] = acc

        for cp in (cp3a, cp3b, cp3c, cp3d):
            cp.wait()
        for i in range(_CHUNK // _L):
            s = pl.ds(i * _L, _L)
            cv[s] = bg[s] + tbg[s]
        pltpu.sync_copy(dotv, dot_out.at[pl.ds(base, _CHUNK)])
        pltpu.sync_copy(cv, c_out.at[pl.ds(base, _CHUNK)])

    return sc_kernel


def _sc_co_fn():
    @functools.partial(
        pl.kernel,
        mesh=_MESH,
        out_type=jax.ShapeDtypeStruct((_B,), jnp.float32),
        scratch_types=[
            pltpu.VMEM((_CHUNK,), jnp.int32),
            pltpu.VMEM((_CHUNK,), jnp.int32),
            pltpu.VMEM((_CHUNK,), jnp.int32),
            pltpu.VMEM((_CHUNK,), jnp.float32),
            pltpu.SemaphoreType.DMA,
        ],
        compiler_params=_SC_PARAMS,
    )
    def sc_kernel(words_hbm, targets_hbm, co_hbm, dep_hbm, co_out,
                  wv, tv, ci, cov, sem):
        del dep_hbm
        base = _worker_base()
        i1 = pltpu.async_copy(words_hbm.at[pl.ds(base, _CHUNK)], wv, sem)
        i2 = pltpu.async_copy(targets_hbm.at[pl.ds(base, _CHUNK)], tv, sem)
        i1.wait()
        i2.wait()
        for i in range(_CHUNK // _L):
            s = pl.ds(i * _L, _L)
            ci[s] = wv[s] * _VOCAB + tv[s]
        pltpu.async_copy(co_hbm.at[ci], cov, sem).wait()
        pltpu.sync_copy(cov, co_out.at[pl.ds(base, _CHUNK)])

    return sc_kernel


def _tc_loss(co2, dot2, c2):
    def body(co_ref, dot_ref, c_ref, out_ref):
        co = co_ref[...]
        dot = dot_ref[...]
        c = c_ref[...]
        w = jnp.minimum(jnp.exp(jnp.log(co / _X_MAX) * _ALPHA), 1.0)
        a = w * dot - jnp.log(co)
        s1 = jnp.sum(a * a)
        s2 = jnp.sum(a * w)
        s3 = jnp.sum(w * w)
        s4 = jnp.sum(c)
        s5 = jnp.sum(c * c)
        out_ref[0, 0] = _B * s1 + 2.0 * s2 * s4 + s3 * s5

    return pl.pallas_call(
        body,
        out_shape=jax.ShapeDtypeStruct((1, 1), jnp.float32),
        out_specs=pl.BlockSpec(memory_space=pltpu.SMEM),
    )(co2, dot2, c2)


def kernel(words, target_words, W, b, tilde_W, tilde_b, co_mat):
    wtw = jnp.concatenate([W, tilde_W], axis=0)
    btb = jnp.concatenate([b, tilde_b], axis=0).reshape(-1)
    co_flat = co_mat.reshape(-1)
    dot, c = _sc_embed_fn()(words, target_words, wtw, btb)
    co = _sc_co_fn()(words, target_words, co_flat, dot)
    loss = _tc_loss(co.reshape(_NW, _CHUNK), dot.reshape(_NW, _CHUNK),
                    c.reshape(_NW, _CHUNK))
    return loss[0, 0]


# speedup vs baseline: 1.1049x; 1.1049x over previous
import functools
import jax
import jax.numpy as jnp
from jax import lax
from jax.experimental import pallas as pl
from jax.experimental.pallas import tpu as pltpu
from jax.experimental.pallas import tpu_sc as plsc

_VOCAB = 1000
_DIM = 64
_B = 4096
_X_MAX = 100.0
_ALPHA = 0.75

_NC = 2
_NS = 16
_NW = _NC * _NS
_CHUNK = _B // _NW
_HALF = _CHUNK // 2
_L = 16

_SC_PARAMS = pltpu.CompilerParams(
    use_tc_tiling_on_sc=False,
    needs_layout_passes=False,
)
_MESH = plsc.VectorSubcoreMesh(core_axis_name="c", subcore_axis_name="s")


def _worker_base():
    wid = lax.axis_index("s") * _NC + lax.axis_index("c")
    return wid * _CHUNK


def _sc_embed_fn():
    @functools.partial(
        pl.kernel,
        mesh=_MESH,
        out_type=(
            jax.ShapeDtypeStruct((_B,), jnp.float32),
            jax.ShapeDtypeStruct((_B,), jnp.float32),
        ),
        scratch_types=[
            pltpu.VMEM((_HALF,), jnp.int32),
            pltpu.VMEM((_HALF,), jnp.int32),
            pltpu.VMEM((_HALF,), jnp.int32),
            pltpu.VMEM((_HALF,), jnp.int32),
            pltpu.VMEM((_HALF, _DIM), jnp.float32),
            pltpu.VMEM((_HALF, _DIM), jnp.float32),
            pltpu.VMEM((_HALF, _DIM), jnp.float32),
            pltpu.VMEM((_HALF, _DIM), jnp.float32),
            pltpu.VMEM((_CHUNK,), jnp.float32),
            pltpu.VMEM((_CHUNK,), jnp.float32),
            pltpu.VMEM((_CHUNK * _L,), jnp.float32),
            pltpu.VMEM((_CHUNK,), jnp.float32),
            pltpu.VMEM((_CHUNK,), jnp.float32),
            pltpu.SemaphoreType.DMA,
            pltpu.SemaphoreType.DMA,
            pltpu.SemaphoreType.DMA,
            pltpu.SemaphoreType.DMA,
        ],
        compiler_params=_SC_PARAMS,
    )
    def sc_kernel(words_hbm, targets_hbm, w_hbm, tw_hbm, b_hbm, tb_hbm,
                  dot_out, c_out,
                  wv1, wv2, tv1, tv2, ew1, ew2, etw1, etw2, bg, tbg,
                  qflat, dotv, cv, sem0, sem1, sem2, sem3):
        base = _worker_base()
        i1 = pltpu.async_copy(words_hbm.at[pl.ds(base, _HALF)], wv1, sem0)
        i2 = pltpu.async_copy(targets_hbm.at[pl.ds(base, _HALF)], tv1, sem0)
        i3 = pltpu.async_copy(
            words_hbm.at[pl.ds(base + _HALF, _HALF)], wv2, sem0)
        i4 = pltpu.async_copy(
            targets_hbm.at[pl.ds(base + _HALF, _HALF)], tv2, sem0)
        for cp in (i1, i2, i3, i4):
            cp.wait()
        cp1a = pltpu.async_copy(w_hbm.at[wv1], ew1, sem1)
        cp1b = pltpu.async_copy(tw_hbm.at[tv1], etw1, sem1)
        cp2a = pltpu.async_copy(w_hbm.at[wv2], ew2, sem2)
        cp2b = pltpu.async_copy(tw_hbm.at[tv2], etw2, sem2)
        cp3a = pltpu.async_copy(b_hbm.at[wv1], bg.at[pl.ds(0, _HALF)], sem3)
        cp3b = pltpu.async_copy(
            b_hbm.at[wv2], bg.at[pl.ds(_HALF, _HALF)], sem3)
        cp3c = pltpu.async_copy(
            tb_hbm.at[tv1], tbg.at[pl.ds(0, _HALF)], sem3)
        cp3d = pltpu.async_copy(
            tb_hbm.at[tv2], tbg.at[pl.ds(_HALF, _HALF)], sem3)

        def fold_half(ew, etw, qbase):
            @plsc.parallel_loop(0, _HALF, unroll=4)
            def fold_body(k):
                s = pl.ds(0, _L)
                acc = ew[k, s] * etw[k, s]
                for j in range(1, _DIM // _L):
                    s = pl.ds(j * _L, _L)
                    acc = acc + ew[k, s] * etw[k, s]
                qflat[pl.ds(qbase + k * _L, _L)] = acc

        cp1a.wait()
        cp1b.wait()
        fold_half(ew1, etw1, 0)
        cp2a.wait()
        cp2b.wait()
        fold_half(ew2, etw2, _HALF * _L)

        lanebase = lax.iota(jnp.int32, _L) * _L

        @plsc.parallel_loop(0, _CHUNK // _L)
        def red_group(g):
            gb = g * (_L * _L)
            acc = plsc.load_gather(qflat, [lanebase + gb])
            for d in range(1, _L):
                acc = acc + plsc.load_gather(qflat, [lanebase + (gb + d)])
            dotv[pl.ds(g * _L, _L)] = acc

        for cp in (cp3a, cp3b, cp3c, cp3d):
            cp.wait()
        for i in range(_CHUNK // _L):
            s = pl.ds(i * _L, _L)
            cv[s] = bg[s] + tbg[s]
        pltpu.sync_copy(dotv, dot_out.at[pl.ds(base, _CHUNK)])
        pltpu.sync_copy(cv, c_out.at[pl.ds(base, _CHUNK)])

    return sc_kernel


def _sc_co_fn():
    @functools.partial(
        pl.kernel,
        mesh=_MESH,
        out_type=jax.ShapeDtypeStruct((_B,), jnp.float32),
        scratch_types=[
            pltpu.VMEM((_CHUNK,), jnp.int32),
            pltpu.VMEM((_CHUNK,), jnp.int32),
            pltpu.VMEM((_CHUNK,), jnp.int32),
            pltpu.VMEM((_CHUNK,), jnp.float32),
            pltpu.SemaphoreType.DMA,
        ],
        compiler_params=_SC_PARAMS,
    )
    def sc_kernel(words_hbm, targets_hbm, co_hbm, dep_hbm, co_out,
                  wv, tv, ci, cov, sem):
        del dep_hbm
        base = _worker_base()
        i1 = pltpu.async_copy(words_hbm.at[pl.ds(base, _CHUNK)], wv, sem)
        i2 = pltpu.async_copy(targets_hbm.at[pl.ds(base, _CHUNK)], tv, sem)
        i1.wait()
        i2.wait()
        for i in range(_CHUNK // _L):
            s = pl.ds(i * _L, _L)
            ci[s] = wv[s] * _VOCAB + tv[s]
        pltpu.async_copy(co_hbm.at[ci], cov, sem).wait()
        pltpu.sync_copy(cov, co_out.at[pl.ds(base, _CHUNK)])

    return sc_kernel


def _tc_loss(co2, dot2, c2):
    def body(co_ref, dot_ref, c_ref, out_ref):
        co = co_ref[...]
        dot = dot_ref[...]
        c = c_ref[...]
        w = jnp.minimum(jnp.exp(jnp.log(co / _X_MAX) * _ALPHA), 1.0)
        a = w * dot - jnp.log(co)
        s1 = jnp.sum(a * a)
        s2 = jnp.sum(a * w)
        s3 = jnp.sum(w * w)
        s4 = jnp.sum(c)
        s5 = jnp.sum(c * c)
        out_ref[0, 0] = _B * s1 + 2.0 * s2 * s4 + s3 * s5

    return pl.pallas_call(
        body,
        out_shape=jax.ShapeDtypeStruct((1, 1), jnp.float32),
        out_specs=pl.BlockSpec(memory_space=pltpu.SMEM),
    )(co2, dot2, c2)


def kernel(words, target_words, W, b, tilde_W, tilde_b, co_mat):
    b_flat = b.reshape(-1)
    tb_flat = tilde_b.reshape(-1)
    co_flat = co_mat.reshape(-1)
    dot, c = _sc_embed_fn()(
        words, target_words, W, tilde_W, b_flat, tb_flat)
    co = _sc_co_fn()(words, target_words, co_flat, dot)
    loss = _tc_loss(co.reshape(_NW, _CHUNK), dot.reshape(_NW, _CHUNK),
                    c.reshape(_NW, _CHUNK))
    return loss[0, 0]


# speedup vs baseline: 1.1112x; 1.0057x over previous
import functools
import jax
import jax.numpy as jnp
from jax import lax
from jax.experimental import pallas as pl
from jax.experimental.pallas import tpu as pltpu
from jax.experimental.pallas import tpu_sc as plsc

_VOCAB = 1000
_DIM = 64
_B = 4096
_X_MAX = 100.0
_ALPHA = 0.75

_NC = 2
_NS = 16
_NW = _NC * _NS
_CHUNK = _B // _NW
_HALF = _CHUNK // 2
_L = 16

_SC_PARAMS = pltpu.CompilerParams(
    use_tc_tiling_on_sc=False,
    needs_layout_passes=False,
)
_MESH = plsc.VectorSubcoreMesh(core_axis_name="c", subcore_axis_name="s")


def _worker_base():
    wid = lax.axis_index("s") * _NC + lax.axis_index("c")
    return wid * _CHUNK


def _sc_embed_fn():
    @functools.partial(
        pl.kernel,
        mesh=_MESH,
        out_type=(
            jax.ShapeDtypeStruct((_B,), jnp.float32),
            jax.ShapeDtypeStruct((_B,), jnp.float32),
        ),
        scratch_types=[
            pltpu.VMEM((_HALF,), jnp.int32),
            pltpu.VMEM((_HALF,), jnp.int32),
            pltpu.VMEM((_HALF,), jnp.int32),
            pltpu.VMEM((_HALF,), jnp.int32),
            pltpu.VMEM((_HALF, _DIM), jnp.float32),
            pltpu.VMEM((_HALF, _DIM), jnp.float32),
            pltpu.VMEM((_HALF, _DIM), jnp.float32),
            pltpu.VMEM((_HALF, _DIM), jnp.float32),
            pltpu.VMEM((_CHUNK,), jnp.float32),
            pltpu.VMEM((_CHUNK,), jnp.float32),
            pltpu.VMEM((_CHUNK * _L,), jnp.float32),
            pltpu.VMEM((_CHUNK,), jnp.float32),
            pltpu.VMEM((_CHUNK,), jnp.float32),
            pltpu.SemaphoreType.DMA,
            pltpu.SemaphoreType.DMA,
            pltpu.SemaphoreType.DMA,
            pltpu.SemaphoreType.DMA,
        ],
        compiler_params=_SC_PARAMS,
    )
    def sc_kernel(words_hbm, targets_hbm, w_hbm, tw_hbm, b_hbm, tb_hbm,
                  dot_out, c_out,
                  wv1, wv2, tv1, tv2, ew1, ew2, etw1, etw2, bg, tbg,
                  qflat, dotv, cv, sem0, sem1, sem2, sem3):
        base = _worker_base()
        i1 = pltpu.async_copy(words_hbm.at[pl.ds(base, _HALF)], wv1, sem0)
        i2 = pltpu.async_copy(targets_hbm.at[pl.ds(base, _HALF)], tv1, sem0)
        i3 = pltpu.async_copy(
            words_hbm.at[pl.ds(base + _HALF, _HALF)], wv2, sem0)
        i4 = pltpu.async_copy(
            targets_hbm.at[pl.ds(base + _HALF, _HALF)], tv2, sem0)
        i1.wait()
        cp1a = pltpu.async_copy(w_hbm.at[wv1], ew1, sem1)
        i2.wait()
        cp1b = pltpu.async_copy(tw_hbm.at[tv1], etw1, sem1)
        i3.wait()
        cp2a = pltpu.async_copy(w_hbm.at[wv2], ew2, sem2)
        i4.wait()
        cp2b = pltpu.async_copy(tw_hbm.at[tv2], etw2, sem2)
        cp3a = pltpu.async_copy(b_hbm.at[wv1], bg.at[pl.ds(0, _HALF)], sem3)
        cp3b = pltpu.async_copy(
            b_hbm.at[wv2], bg.at[pl.ds(_HALF, _HALF)], sem3)
        cp3c = pltpu.async_copy(
            tb_hbm.at[tv1], tbg.at[pl.ds(0, _HALF)], sem3)
        cp3d = pltpu.async_copy(
            tb_hbm.at[tv2], tbg.at[pl.ds(_HALF, _HALF)], sem3)

        def fold_half(ew, etw, qbase):
            @plsc.parallel_loop(0, _HALF, unroll=4)
            def fold_body(k):
                s = pl.ds(0, _L)
                acc = ew[k, s] * etw[k, s]
                for j in range(1, _DIM // _L):
                    s = pl.ds(j * _L, _L)
                    acc = acc + ew[k, s] * etw[k, s]
                qflat[pl.ds(qbase + k * _L, _L)] = acc

        cp1a.wait()
        cp1b.wait()
        fold_half(ew1, etw1, 0)
        cp2a.wait()
        cp2b.wait()
        fold_half(ew2, etw2, _HALF * _L)

        lanebase = lax.iota(jnp.int32, _L) * _L

        @plsc.parallel_loop(0, _CHUNK // _L)
        def red_group(g):
            gb = g * (_L * _L)
            acc = plsc.load_gather(qflat, [lanebase + gb])
            for d in range(1, _L):
                acc = acc + plsc.load_gather(qflat, [lanebase + (gb + d)])
            dotv[pl.ds(g * _L, _L)] = acc

        for cp in (cp3a, cp3b, cp3c, cp3d):
            cp.wait()
        for i in range(_CHUNK // _L):
            s = pl.ds(i * _L, _L)
            cv[s] = bg[s] + tbg[s]
        o1 = pltpu.async_copy(dotv, dot_out.at[pl.ds(base, _CHUNK)], sem0)
        o2 = pltpu.async_copy(cv, c_out.at[pl.ds(base, _CHUNK)], sem0)
        o1.wait()
        o2.wait()

    return sc_kernel


def _sc_co_fn():
    @functools.partial(
        pl.kernel,
        mesh=_MESH,
        out_type=jax.ShapeDtypeStruct((_B,), jnp.float32),
        scratch_types=[
            pltpu.VMEM((_CHUNK,), jnp.int32),
            pltpu.VMEM((_CHUNK,), jnp.int32),
            pltpu.VMEM((_CHUNK,), jnp.int32),
            pltpu.VMEM((_CHUNK,), jnp.float32),
            pltpu.SemaphoreType.DMA,
        ],
        compiler_params=_SC_PARAMS,
    )
    def sc_kernel(words_hbm, targets_hbm, co_hbm, dep_hbm, co_out,
                  wv, tv, ci, cov, sem):
        del dep_hbm
        base = _worker_base()
        i1 = pltpu.async_copy(words_hbm.at[pl.ds(base, _CHUNK)], wv, sem)
        i2 = pltpu.async_copy(targets_hbm.at[pl.ds(base, _CHUNK)], tv, sem)
        i1.wait()
        i2.wait()
        for i in range(_CHUNK // _L):
            s = pl.ds(i * _L, _L)
            ci[s] = wv[s] * _VOCAB + tv[s]
        pltpu.async_copy(co_hbm.at[ci], cov, sem).wait()
        pltpu.sync_copy(cov, co_out.at[pl.ds(base, _CHUNK)])

    return sc_kernel


def _tc_loss(co2, dot2, c2):
    def body(co_ref, dot_ref, c_ref, out_ref):
        co = co_ref[...]
        dot = dot_ref[...]
        c = c_ref[...]
        w = jnp.minimum(jnp.exp(jnp.log(co / _X_MAX) * _ALPHA), 1.0)
        a = w * dot - jnp.log(co)
        s1 = jnp.sum(a * a)
        s2 = jnp.sum(a * w)
        s3 = jnp.sum(w * w)
        s4 = jnp.sum(c)
        s5 = jnp.sum(c * c)
        out_ref[0, 0] = _B * s1 + 2.0 * s2 * s4 + s3 * s5

    return pl.pallas_call(
        body,
        out_shape=jax.ShapeDtypeStruct((1, 1), jnp.float32),
        out_specs=pl.BlockSpec(memory_space=pltpu.SMEM),
    )(co2, dot2, c2)


def kernel(words, target_words, W, b, tilde_W, tilde_b, co_mat):
    b_flat = b.reshape(-1)
    tb_flat = tilde_b.reshape(-1)
    co_flat = co_mat.reshape(-1)
    dot, c = _sc_embed_fn()(
        words, target_words, W, tilde_W, b_flat, tb_flat)
    co = _sc_co_fn()(words, target_words, co_flat, dot)
    loss = _tc_loss(co.reshape(_NW, _CHUNK), dot.reshape(_NW, _CHUNK),
                    c.reshape(_NW, _CHUNK))
    return loss[0, 0]
